# baseline (speedup 1.0000x reference)
import functools

import jax
import jax.numpy as jnp
from jax import lax
from jax.experimental import pallas as pl
from jax.experimental.pallas import tpu as pltpu
from jax.experimental.pallas import tpu_sc as plsc

B = 4
E = 16
H = 16
S = 512
D = 1024
HB = 2


def _tc_reduce_and_logits(mh, mm, wr, br2, wn, bn2, eps):
    grid = (B, H // HB)
    nh = H // HB

    def body(mh_ref, mm_ref, wr_ref, br_ref, wn_ref, bn_ref, eps_ref,
             noisy_ref, mm16_ref, acc_ref):
        b = pl.program_id(0)
        h = pl.program_id(1)

        @pl.when((b == 0) & (h == 0))
        def _():
            acc_ref[...] = jnp.zeros_like(acc_ref)

        blk = mh_ref[0].reshape(HB * S, D)
        ones = jnp.ones((1, HB * S), jnp.float32)
        s = lax.dot_general(ones, blk, (((1,), (0,)), ((), ())),
                            preferred_element_type=jnp.float32)
        rows = lax.broadcasted_iota(jnp.int32, (B, D), 0)
        acc_ref[...] += jnp.where(rows == b, s, 0.0)

        @pl.when((b == B - 1) & (h == nh - 1))
        def _():
            x = acc_ref[...] * (1.0 / (H * S))
            r = lax.dot_general(x, wr_ref[...], (((1,), (1,)), ((), ())),
                                preferred_element_type=jnp.float32) + br_ref[...]
            logits = jax.nn.softmax(r, axis=-1)
            n = lax.dot_general(x, wn_ref[...], (((1,), (1,)), ((), ())),
                                preferred_element_type=jnp.float32) + bn_ref[...]
            noise = jax.nn.softmax(eps_ref[...] * jax.nn.softplus(n), axis=-1)
            noisy_ref[...] = logits + noise
            mmf = mm_ref[...].astype(jnp.float32)
            mm16_ref[...] = jnp.concatenate(
                [mmf, jnp.zeros((1, E - B), jnp.float32)], axis=1)

    return pl.pallas_call(
        body,
        grid=grid,
        in_specs=[
            pl.BlockSpec((1, HB, S, D), lambda b, h: (b, h, 0, 0)),
            pl.BlockSpec((1, B), lambda b, h: (0, 0)),
            pl.BlockSpec((E, D), lambda b, h: (0, 0)),
            pl.BlockSpec((1, E), lambda b, h: (0, 0)),
            pl.BlockSpec((E, D), lambda b, h: (0, 0)),
            pl.BlockSpec((1, E), lambda b, h: (0, 0)),
            pl.BlockSpec((B, E), lambda b, h: (0, 0)),
        ],
        out_specs=[
            pl.BlockSpec((B, E), lambda b, h: (0, 0)),
            pl.BlockSpec((1, E), lambda b, h: (0, 0)),
        ],
        out_shape=[
            jax.ShapeDtypeStruct((B, E), jnp.float32),
            jax.ShapeDtypeStruct((1, E), jnp.float32),
        ],
        scratch_shapes=[pltpu.VMEM((B, D), jnp.float32)],
        compiler_params=pltpu.CompilerParams(
            dimension_semantics=("arbitrary", "arbitrary")),
    )(mh, mm, wr, br2, wn, bn2, eps)


def _shuffle(x, lanes, s):
    return x.at[lanes ^ s].get(mode="promise_in_bounds")


def _allmax(x, lanes):
    for s in (8, 4, 2, 1):
        x = jnp.maximum(x, _shuffle(x, lanes, s))
    return x


def _allsum(x, lanes):
    for s in (8, 4, 2, 1):
        x = x + _shuffle(x, lanes, s)
    return x


def _sc_route(noisy, mm16):
    mesh = plsc.VectorSubcoreMesh(core_axis_name="c", subcore_axis_name="s",
                                  num_cores=1, num_subcores=B)

    @functools.partial(
        pl.kernel,
        mesh=mesh,
        out_type=jax.ShapeDtypeStruct((B, E), jnp.float32),
        scratch_types=[
            pltpu.VMEM((E,), jnp.float32),
            pltpu.VMEM((E,), jnp.float32),
            pltpu.VMEM((E,), jnp.float32),
        ],
    )
    def k(noisy_hbm, mm_hbm, out_hbm, nv_v, mm_v, out_v):
        wid = lax.axis_index("s")

        @pl.when(wid < B)
        def _():
            b = wid
            pltpu.sync_copy(noisy_hbm.at[b], nv_v)
            pltpu.sync_copy(mm_hbm.at[0], mm_v)
            nv = nv_v[...]
            mm = mm_v[...]
            lanes = lax.iota(jnp.int32, 16)
            kb = mm.at[jnp.full((E,), b, jnp.int32)].get(
                mode="promise_in_bounds")
            maxk = _allmax(mm, lanes)
            rank = jnp.zeros((E,), jnp.float32)
            for j in range(E):
                bj = nv.at[jnp.full((E,), j, jnp.int32)].get(
                    mode="promise_in_bounds")
                beats = (bj > nv) | ((bj == nv) & (j < lanes))
                rank = rank + jnp.where(beats, 1.0, 0.0)
            member = rank < kb
            sparse = jnp.where(member, nv, 0.0)
            zero0 = jnp.logical_and(lanes == 0, kb < maxk)
            sparse = jnp.where(zero0, 0.0, sparse)
            m = _allmax(sparse, lanes)
            ex = jnp.exp(sparse - m)
            ssum = _allsum(ex, lanes)
            out_v[...] = ex / ssum
            pltpu.sync_copy(out_v, out_hbm.at[b])

    return k(noisy, mm16)


def kernel(mh_output, mis_mask, W_route, b_route, W_noise, b_noise):
    eps = jax.random.normal(jax.random.key(42), (B, E), dtype=jnp.float32)
    noisy, mm16 = _tc_reduce_and_logits(
        mh_output, mis_mask.reshape(1, B), W_route, b_route.reshape(1, E),
        W_noise, b_noise.reshape(1, E), eps)
    router = _sc_route(noisy, mm16)
    return router, noisy


# speedup vs baseline: 1.0763x; 1.0763x over previous
import functools

import jax
import jax.numpy as jnp
from jax import lax
from jax.experimental import pallas as pl
from jax.experimental.pallas import tpu as pltpu
from jax.experimental.pallas import tpu_sc as plsc

B = 4
E = 16
H = 16
S = 512
D = 1024
HB = 4


def _tc_reduce_and_logits(mh, mm, wr, br2, wn, bn2, eps):
    grid = (B, H // HB)
    nh = H // HB

    def body(mh_ref, mm_ref, wr_ref, br_ref, wn_ref, bn_ref, eps_ref,
             noisy_ref, combo_ref, acc_ref):
        b = pl.program_id(0)
        h = pl.program_id(1)

        @pl.when((b == 0) & (h == 0))
        def _():
            acc_ref[...] = jnp.zeros_like(acc_ref)

        blk = mh_ref[0].reshape(HB * S, D)
        ones = jnp.ones((1, HB * S), jnp.float32)
        s = lax.dot_general(ones, blk, (((1,), (0,)), ((), ())),
                            preferred_element_type=jnp.float32)
        rows = lax.broadcasted_iota(jnp.int32, (B, D), 0)
        acc_ref[...] += jnp.where(rows == b, s, 0.0)

        @pl.when((b == B - 1) & (h == nh - 1))
        def _():
            x = acc_ref[...] * (1.0 / (H * S))
            r = lax.dot_general(x, wr_ref[...], (((1,), (1,)), ((), ())),
                                preferred_element_type=jnp.float32) + br_ref[...]
            logits = jax.nn.softmax(r, axis=-1)
            n = lax.dot_general(x, wn_ref[...], (((1,), (1,)), ((), ())),
                                preferred_element_type=jnp.float32) + bn_ref[...]
            noise = jax.nn.softmax(eps_ref[...] * jax.nn.softplus(n), axis=-1)
            noisy = logits + noise
            noisy_ref[...] = noisy
            mmf = mm_ref[...].astype(jnp.float32)
            mm16 = jnp.concatenate(
                [mmf, jnp.zeros((1, E - B), jnp.float32)], axis=1)
            combo_ref[...] = jnp.concatenate(
                [noisy, mm16, jnp.zeros((8 - B - 1, E), jnp.float32)], axis=0)

    return pl.pallas_call(
        body,
        grid=grid,
        in_specs=[
            pl.BlockSpec((1, HB, S, D), lambda b, h: (b, h, 0, 0)),
            pl.BlockSpec((1, B), lambda b, h: (0, 0)),
            pl.BlockSpec((E, D), lambda b, h: (0, 0)),
            pl.BlockSpec((1, E), lambda b, h: (0, 0)),
            pl.BlockSpec((E, D), lambda b, h: (0, 0)),
            pl.BlockSpec((1, E), lambda b, h: (0, 0)),
            pl.BlockSpec((B, E), lambda b, h: (0, 0)),
        ],
        out_specs=[
            pl.BlockSpec((B, E), lambda b, h: (0, 0)),
            pl.BlockSpec((8, E), lambda b, h: (0, 0)),
        ],
        out_shape=[
            jax.ShapeDtypeStruct((B, E), jnp.float32),
            jax.ShapeDtypeStruct((8, E), jnp.float32),
        ],
        scratch_shapes=[pltpu.VMEM((B, D), jnp.float32)],
        compiler_params=pltpu.CompilerParams(
            dimension_semantics=("arbitrary", "arbitrary")),
    )(mh, mm, wr, br2, wn, bn2, eps)


def _shuffle(x, lanes, s):
    return x.at[lanes ^ s].get(mode="promise_in_bounds")


def _allmax(x, lanes):
    for s in (8, 4, 2, 1):
        x = jnp.maximum(x, _shuffle(x, lanes, s))
    return x


def _allsum(x, lanes):
    for s in (8, 4, 2, 1):
        x = x + _shuffle(x, lanes, s)
    return x


def _sc_route(combo):
    mesh = plsc.VectorSubcoreMesh(core_axis_name="c", subcore_axis_name="s",
                                  num_cores=1, num_subcores=1)

    @functools.partial(
        pl.kernel,
        mesh=mesh,
        out_type=jax.ShapeDtypeStruct((B, E), jnp.float32),
        scratch_types=[
            pltpu.VMEM((8, E), jnp.float32),
            pltpu.VMEM((B, E), jnp.float32),
        ],
    )
    def k(combo_hbm, out_hbm, comb_v, out_v):
        pltpu.sync_copy(combo_hbm, comb_v)
        mm = comb_v[B]
        lanes = lax.iota(jnp.int32, 16)
        maxk = _allmax(mm, lanes)
        for b in range(B):
            nv = comb_v[b]
            kb = mm.at[jnp.full((E,), b, jnp.int32)].get(
                mode="promise_in_bounds")
            rank = jnp.zeros((E,), jnp.float32)
            for j in range(E):
                bj = nv.at[jnp.full((E,), j, jnp.int32)].get(
                    mode="promise_in_bounds")
                beats = (bj > nv) | ((bj == nv) & (j < lanes))
                rank = rank + jnp.where(beats, 1.0, 0.0)
            member = rank < kb
            sparse = jnp.where(member, nv, 0.0)
            zero0 = jnp.logical_and(lanes == 0, kb < maxk)
            sparse = jnp.where(zero0, 0.0, sparse)
            m = _allmax(sparse, lanes)
            ex = jnp.exp(sparse - m)
            ssum = _allsum(ex, lanes)
            out_v[b] = ex / ssum
        pltpu.sync_copy(out_v, out_hbm)

    return k(combo)


def kernel(mh_output, mis_mask, W_route, b_route, W_noise, b_noise):
    eps = jax.random.normal(jax.random.key(42), (B, E), dtype=jnp.float32)
    noisy, combo = _tc_reduce_and_logits(
        mh_output, mis_mask.reshape(1, B), W_route, b_route.reshape(1, E),
        W_noise, b_noise.reshape(1, E), eps)
    router = _sc_route(combo)
    return router, noisy
